# Initial kernel scaffold; baseline (speedup 1.0000x reference)
import jax
import jax.numpy as jnp
from jax.experimental import pallas as pl


def kernel(x, edge_index, edge_weight, W_fc, W_attn):
    raise NotImplementedError("write your pallas kernel here")


# speedup vs baseline: 10.1578x; 10.1578x over previous
import functools

import jax
import jax.numpy as jnp
from jax import lax
from jax.experimental import pallas as pl
from jax.experimental.pallas import tpu as pltpu
from jax.experimental.pallas import tpu_sc as plsc

N = 10000
E = 320000
D = 128
NPAD = 10240
NC = 2
NS = 16
NW = NC * NS
CH = 128
EPW = 10240
EP = NW * EPW
NCH = EPW // CH
GC = 8
SL = NPAD // NS


def _mesh():
    return plsc.VectorSubcoreMesh(
        core_axis_name="c", subcore_axis_name="s", num_cores=NC, num_subcores=NS
    )


def _project(x, W_fc, Wpair):
    def body(x_ref, wf_ref, wp_ref, z_ref, a_ref):
        zb = jnp.dot(x_ref[...], wf_ref[...], preferred_element_type=jnp.float32)
        z_ref[...] = zb
        a_ref[...] = jnp.dot(zb, wp_ref[...], preferred_element_type=jnp.float32)

    return pl.pallas_call(
        body,
        grid=(10,),
        in_specs=[
            pl.BlockSpec((1000, D), lambda i: (i, 0)),
            pl.BlockSpec((D, D), lambda i: (0, 0)),
            pl.BlockSpec((D, D), lambda i: (0, 0)),
        ],
        out_specs=[
            pl.BlockSpec((1000, D), lambda i: (i, 0)),
            pl.BlockSpec((1000, D), lambda i: (i, 0)),
        ],
        out_shape=[
            jax.ShapeDtypeStruct((N, D), jnp.float32),
            jax.ShapeDtypeStruct((N, D), jnp.float32),
        ],
    )(x, W_fc, Wpair)


@functools.partial(
    pl.kernel,
    out_type=[
        jax.ShapeDtypeStruct((EP,), jnp.float32),
        jax.ShapeDtypeStruct((NC * NPAD,), jnp.float32),
    ],
    mesh=_mesh(),
    compiler_params=pltpu.CompilerParams(needs_layout_passes=False),
    scratch_types=[
        pltpu.VMEM((NPAD,), jnp.float32),
        pltpu.VMEM((NPAD,), jnp.float32),
        pltpu.VMEM((EPW,), jnp.int32),
        pltpu.VMEM((EPW,), jnp.int32),
        pltpu.VMEM((EPW,), jnp.float32),
        pltpu.VMEM((EPW,), jnp.float32),
        pltpu.VMEM((NPAD,), jnp.float32),
        pltpu.VMEM((NS, SL), jnp.float32),
        pltpu.VMEM_SHARED((NS * NPAD,), jnp.float32),
    ],
)
def _score(a_s_h, a_d_h, src_h, dst_h, ew_h, eexp_h, den2_h,
           a_s_v, a_d_v, src_v, dst_v, ew_v, eexp_v, den_v, red_v, sh_d):
    cid = lax.axis_index("c")
    sid = lax.axis_index("s")
    wid = cid * NS + sid
    base = wid * EPW

    pltpu.sync_copy(a_s_h, a_s_v)
    pltpu.sync_copy(a_d_h, a_d_v)
    pltpu.sync_copy(src_h.at[pl.ds(base, EPW)], src_v)
    pltpu.sync_copy(dst_h.at[pl.ds(base, EPW)], dst_v)
    pltpu.sync_copy(ew_h.at[pl.ds(base, EPW)], ew_v)

    zeros = jnp.zeros((16,), jnp.float32)

    def zero_body(i, _):
        den_v[pl.ds(i * 16, 16)] = zeros
        return 0

    lax.fori_loop(0, NPAD // 16, zero_body, 0)

    def edge_body(i, _):
        s16 = src_v[pl.ds(i * 16, 16)]
        d16 = dst_v[pl.ds(i * 16, 16)]
        w16 = ew_v[pl.ds(i * 16, 16)]
        t = plsc.load_gather(a_s_v, [s16]) + plsc.load_gather(a_d_v, [d16])
        e = jnp.where(t >= 0.0, t, t * 0.01) * w16
        p = jnp.exp(e)
        eexp_v[pl.ds(i * 16, 16)] = p
        plsc.addupdate_scatter(den_v, [d16], p)
        return 0

    lax.fori_loop(0, EPW // 16, edge_body, 0)
    pltpu.sync_copy(eexp_v, eexp_h.at[pl.ds(base, EPW)])

    pltpu.sync_copy(den_v, sh_d.at[pl.ds(sid * NPAD, NPAD)])
    plsc.subcore_barrier()
    for r in range(NS):
        pltpu.sync_copy(sh_d.at[pl.ds(r * NPAD + sid * SL, SL)], red_v.at[r])

    def red_body(g, _):
        acc = red_v[0, pl.ds(g * 16, 16)]
        for r in range(1, NS):
            acc = acc + red_v[r, pl.ds(g * 16, 16)]
        den_v[pl.ds(g * 16, 16)] = acc
        return 0

    lax.fori_loop(0, SL // 16, red_body, 0)
    pltpu.sync_copy(
        den_v.at[pl.ds(0, SL)], den2_h.at[pl.ds(cid * NPAD + sid * SL, SL)]
    )


@functools.partial(
    pl.kernel,
    out_type=jax.ShapeDtypeStruct((NC, NPAD, D), jnp.float32),
    mesh=_mesh(),
    compiler_params=pltpu.CompilerParams(needs_layout_passes=False),
    scratch_types=[
        pltpu.VMEM((GC, CH), jnp.int32),
        pltpu.VMEM((GC, CH), jnp.int32),
        pltpu.VMEM((GC, CH), jnp.float32),
        pltpu.VMEM((NPAD,), jnp.float32),
        pltpu.VMEM((NPAD,), jnp.float32),
        pltpu.VMEM((CH,), jnp.float32),
        pltpu.VMEM((CH, D), jnp.float32),
        pltpu.VMEM_SHARED((NPAD, D), jnp.float32),
        pltpu.SemaphoreType.DMA,
    ],
)
def _aggregate(src2_h, dst2_h, eexp2_h, den2_h, z_h, out_h,
               src_v, dst_v, eexp_v, den_v, tmp_v, alpha_v, rows_v, h_sh, sem):
    cid = lax.axis_index("c")
    sid = lax.axis_index("s")
    wid = cid * NS + sid
    rbase = wid * NCH

    pltpu.sync_copy(den2_h.at[pl.ds(0, NPAD)], den_v)
    pltpu.sync_copy(den2_h.at[pl.ds(NPAD, NPAD)], tmp_v)

    def den_body(g, _):
        den_v[pl.ds(g * 16, 16)] = (
            den_v[pl.ds(g * 16, 16)] + tmp_v[pl.ds(g * 16, 16)] + 1e-9
        )
        return 0

    lax.fori_loop(0, NPAD // 16, den_body, 0)

    zeros = jnp.zeros((16,), jnp.float32)

    def zrow_body(j, _):
        for q in range(D // 16):
            rows_v[j, pl.ds(q * 16, 16)] = zeros
        return 0

    lax.fori_loop(0, CH, zrow_body, 0)

    def zcopy_body(m, _):
        pltpu.sync_copy(rows_v, h_sh.at[pl.ds(sid * SL + m * CH, CH)])
        return 0

    lax.fori_loop(0, SL // CH, zcopy_body, 0)
    plsc.subcore_barrier()

    def group_body(g, _):
        gb = rbase + g * GC
        pltpu.sync_copy(src2_h.at[pl.ds(gb, GC)], src_v)
        pltpu.sync_copy(dst2_h.at[pl.ds(gb, GC)], dst_v)
        pltpu.sync_copy(eexp2_h.at[pl.ds(gb, GC)], eexp_v)
        for r in range(GC):
            for k in range(CH // 16):
                d16 = dst_v[r, pl.ds(k * 16, 16)]
                p16 = eexp_v[r, pl.ds(k * 16, 16)]
                dn = plsc.load_gather(den_v, [d16])
                alpha_v[pl.ds(k * 16, 16)] = p16 / dn
            pltpu.async_copy(z_h.at[src_v.at[r]], rows_v, sem).wait()

            def scale_body(j, _):
                av = plsc.load_gather(alpha_v, [jnp.full((16,), j, jnp.int32)])
                for q in range(D // 16):
                    rows_v[j, pl.ds(q * 16, 16)] = (
                        rows_v[j, pl.ds(q * 16, 16)] * av
                    )
                return 0

            lax.fori_loop(0, CH, scale_body, 0)
            pltpu.sync_copy(rows_v, h_sh.at[dst_v.at[r]], add=True)
        return 0

    lax.fori_loop(0, NCH // GC, group_body, 0)
    plsc.subcore_barrier()
    pltpu.sync_copy(
        h_sh.at[pl.ds(sid * SL, SL)], out_h.at[cid, pl.ds(sid * SL, SL)]
    )


def _combine(hp):
    def body(hp_ref, o_ref):
        o_ref[...] = hp_ref[0] + hp_ref[1]

    return pl.pallas_call(
        body,
        grid=(10,),
        in_specs=[pl.BlockSpec((NC, 1000, D), lambda i: (0, i, 0))],
        out_specs=pl.BlockSpec((1000, D), lambda i: (i, 0)),
        out_shape=jax.ShapeDtypeStruct((N, D), jnp.float32),
    )(hp)


def kernel(x, edge_index, edge_weight, W_fc, W_attn):
    src = edge_index[0].astype(jnp.int32)
    dst = edge_index[1].astype(jnp.int32)
    pad = EP - E
    src = jnp.pad(src, (0, pad))
    dst = jnp.pad(dst, (0, pad), constant_values=NPAD - 1)
    ew = jnp.pad(edge_weight, (0, pad))
    wa = W_attn[:, 0]
    Wpair = jnp.zeros((D, D), jnp.float32).at[:, 0].set(wa[:D]).at[:, 1].set(wa[D:])

    z, a = _project(x, W_fc, Wpair)
    a_s = jnp.pad(a[:, 0], (0, NPAD - N))
    a_d = jnp.pad(a[:, 1], (0, NPAD - N))

    eexp, den2 = _score(a_s, a_d, src, dst, ew)

    hp = _aggregate(
        src.reshape(EP // CH, CH),
        dst.reshape(EP // CH, CH),
        eexp.reshape(EP // CH, CH),
        den2,
        z,
    )
    return _combine(hp)


# speedup vs baseline: 13.1374x; 1.2933x over previous
import functools

import jax
import jax.numpy as jnp
from jax import lax
from jax.experimental import pallas as pl
from jax.experimental.pallas import tpu as pltpu
from jax.experimental.pallas import tpu_sc as plsc

N = 10000
E = 320000
D = 128
NPAD = 10240
NC = 2
NS = 16
NW = NC * NS
CH = 128
EPW = 10240
EP = NW * EPW
NCH = EPW // CH
GB = 16
NB = NCH // GB
SL = NPAD // NS


def _mesh():
    return plsc.VectorSubcoreMesh(
        core_axis_name="c", subcore_axis_name="s", num_cores=NC, num_subcores=NS
    )


def _project(x, W_fc, Wpair):
    def body(x_ref, wf_ref, wp_ref, z_ref, a_ref):
        zb = jnp.dot(x_ref[...], wf_ref[...], preferred_element_type=jnp.float32)
        z_ref[...] = zb
        a_ref[...] = jnp.dot(zb, wp_ref[...], preferred_element_type=jnp.float32)

    return pl.pallas_call(
        body,
        grid=(10,),
        in_specs=[
            pl.BlockSpec((1000, D), lambda i: (i, 0)),
            pl.BlockSpec((D, D), lambda i: (0, 0)),
            pl.BlockSpec((D, D), lambda i: (0, 0)),
        ],
        out_specs=[
            pl.BlockSpec((1000, D), lambda i: (i, 0)),
            pl.BlockSpec((1000, D), lambda i: (i, 0)),
        ],
        out_shape=[
            jax.ShapeDtypeStruct((N, D), jnp.float32),
            jax.ShapeDtypeStruct((N, D), jnp.float32),
        ],
    )(x, W_fc, Wpair)


@functools.partial(
    pl.kernel,
    out_type=[
        jax.ShapeDtypeStruct((EP,), jnp.float32),
        jax.ShapeDtypeStruct((NW * NPAD,), jnp.float32),
    ],
    mesh=_mesh(),
    compiler_params=pltpu.CompilerParams(needs_layout_passes=False),
    scratch_types=[
        pltpu.VMEM((NPAD,), jnp.float32),
        pltpu.VMEM((NPAD,), jnp.float32),
        pltpu.VMEM((NPAD,), jnp.float32),
        pltpu.VMEM((EPW,), jnp.int32),
        pltpu.VMEM((EPW,), jnp.int32),
        pltpu.VMEM((EPW,), jnp.float32),
        pltpu.VMEM((EPW,), jnp.float32),
    ],
)
def _score(a_s_h, a_d_h, src_h, dst_h, ew_h, p_h, den_h,
           a_s_v, a_d_v, den_v, src_v, dst_v, ew_v, p_v):
    cid = lax.axis_index("c")
    sid = lax.axis_index("s")
    wid = cid * NS + sid
    base = wid * EPW

    pltpu.sync_copy(a_s_h, a_s_v)
    pltpu.sync_copy(a_d_h, a_d_v)
    pltpu.sync_copy(src_h.at[pl.ds(base, EPW)], src_v)
    pltpu.sync_copy(dst_h.at[pl.ds(base, EPW)], dst_v)
    pltpu.sync_copy(ew_h.at[pl.ds(base, EPW)], ew_v)

    zeros = jnp.zeros((16,), jnp.float32)

    def zden_body(i, _):
        den_v[pl.ds(i * 16, 16)] = zeros
        return 0

    lax.fori_loop(0, NPAD // 16, zden_body, 0)

    def edge_body(i, _):
        s16 = src_v[pl.ds(i * 16, 16)]
        d16 = dst_v[pl.ds(i * 16, 16)]
        w16 = ew_v[pl.ds(i * 16, 16)]
        t = plsc.load_gather(a_s_v, [s16]) + plsc.load_gather(a_d_v, [d16])
        e = jnp.where(t >= 0.0, t, t * 0.01) * w16
        p = jnp.exp(e)
        p_v[pl.ds(i * 16, 16)] = p
        plsc.addupdate_scatter(den_v, [d16], p)
        return 0

    lax.fori_loop(0, EPW // 16, edge_body, 0)
    pltpu.sync_copy(p_v, p_h.at[pl.ds(base, EPW)])
    pltpu.sync_copy(den_v, den_h.at[pl.ds(wid * NPAD, NPAD)])


@functools.partial(
    pl.kernel,
    out_type=jax.ShapeDtypeStruct((NC, NPAD, D), jnp.float32),
    mesh=_mesh(),
    compiler_params=pltpu.CompilerParams(needs_layout_passes=False),
    scratch_types=[
        pltpu.VMEM((GB, CH), jnp.int32),
        pltpu.VMEM((GB, CH), jnp.int32),
        pltpu.VMEM((GB, CH), jnp.float32),
        pltpu.VMEM((CH, D), jnp.float32),
        pltpu.VMEM((CH, D), jnp.float32),
        pltpu.VMEM_SHARED((NPAD, D), jnp.float32),
        pltpu.SemaphoreType.DMA,
        pltpu.SemaphoreType.DMA,
    ],
)
def _aggregate(src2_h, dst2_h, p2_h, z_h, hp_h,
               src_v, dst_v, p_v, rows0_v, rows1_v, h_sh, sem0, sem1):
    cid = lax.axis_index("c")
    sid = lax.axis_index("s")
    wid = cid * NS + sid
    rbase = wid * NCH
    rows = (rows0_v, rows1_v)
    sems = (sem0, sem1)

    zeros = jnp.zeros((16,), jnp.float32)

    def zrow_body(j, _):
        for q in range(D // 16):
            rows0_v[j, pl.ds(q * 16, 16)] = zeros
        return 0

    lax.fori_loop(0, CH, zrow_body, 0)

    def zcopy_body(m, _):
        pltpu.sync_copy(rows0_v, h_sh.at[pl.ds(sid * SL + m * CH, CH)])
        return 0

    lax.fori_loop(0, SL // CH, zcopy_body, 0)
    plsc.subcore_barrier()

    def block_body(b, _):
        bb = rbase + b * GB
        pltpu.sync_copy(src2_h.at[pl.ds(bb, GB)], src_v)
        pltpu.sync_copy(dst2_h.at[pl.ds(bb, GB)], dst_v)
        pltpu.sync_copy(p2_h.at[pl.ds(bb, GB)], p_v)

        descs = [None, None]
        descs[0] = pltpu.async_copy(z_h.at[src_v.at[0]], rows[0], sems[0])
        for r in range(GB):
            cur = r % 2
            if r + 1 < GB:
                nxt = (r + 1) % 2
                descs[nxt] = pltpu.async_copy(
                    z_h.at[src_v.at[r + 1]], rows[nxt], sems[nxt]
                )
            descs[cur].wait()
            rbuf = rows[cur]

            def scale_body(i, _):
                for u in range(2):
                    j = i * 2 + u
                    pv = plsc.load_gather(
                        p_v, [jnp.full((16,), r, jnp.int32),
                              jnp.full((16,), j, jnp.int32)]
                    )
                    for q in range(D // 16):
                        rbuf[j, pl.ds(q * 16, 16)] = (
                            rbuf[j, pl.ds(q * 16, 16)] * pv
                        )
                return 0

            lax.fori_loop(0, CH // 2, scale_body, 0)
            pltpu.sync_copy(rbuf, h_sh.at[dst_v.at[r]], add=True)
        return 0

    lax.fori_loop(0, NB, block_body, 0)
    plsc.subcore_barrier()
    pltpu.sync_copy(
        h_sh.at[pl.ds(sid * SL, SL)], hp_h.at[cid, pl.ds(sid * SL, SL)]
    )


def _combine(hp, den):
    def body(hp_ref, den_ref, o_ref):
        dsum = jnp.sum(den_ref[...], axis=0) + 1e-9
        o_ref[...] = (hp_ref[0] + hp_ref[1]) / dsum[:, None]

    return pl.pallas_call(
        body,
        grid=(8,),
        in_specs=[
            pl.BlockSpec((NC, 1280, D), lambda i: (0, i, 0)),
            pl.BlockSpec((NW, 1280), lambda i: (0, i)),
        ],
        out_specs=pl.BlockSpec((1280, D), lambda i: (i, 0)),
        out_shape=jax.ShapeDtypeStruct((NPAD, D), jnp.float32),
    )(hp, den)


def kernel(x, edge_index, edge_weight, W_fc, W_attn):
    src = edge_index[0].astype(jnp.int32)
    dst = edge_index[1].astype(jnp.int32)
    pad = EP - E
    src = jnp.pad(src, (0, pad))
    dst = jnp.pad(dst, (0, pad), constant_values=NPAD - 1)
    ew = jnp.pad(edge_weight, (0, pad))
    wa = W_attn[:, 0]
    Wpair = jnp.zeros((D, D), jnp.float32).at[:, 0].set(wa[:D]).at[:, 1].set(wa[D:])

    z, a = _project(x, W_fc, Wpair)
    a_s = jnp.pad(a[:, 0], (0, NPAD - N))
    a_d = jnp.pad(a[:, 1], (0, NPAD - N))

    p, den = _score(a_s, a_d, src, dst, ew)

    hp = _aggregate(
        src.reshape(EP // CH, CH),
        dst.reshape(EP // CH, CH),
        p.reshape(EP // CH, CH),
        z,
    )
    return _combine(hp, den.reshape(NW, NPAD))[:N]


# speedup vs baseline: 27.4694x; 2.0909x over previous
import functools

import jax
import jax.numpy as jnp
from jax import lax
from jax.experimental import pallas as pl
from jax.experimental.pallas import tpu as pltpu
from jax.experimental.pallas import tpu_sc as plsc

N = 10000
E = 320000
D = 128
NPAD = 10240
NC = 2
NS = 16
NW = NC * NS
CH = 128
EPW = 10240
EP = NW * EPW
NCH = EPW // CH
GB = 16
NB = NCH // GB
SL = NPAD // NS


def _mesh():
    return plsc.VectorSubcoreMesh(
        core_axis_name="c", subcore_axis_name="s", num_cores=NC, num_subcores=NS
    )


def _project(x, W_fc, Wpair):
    def body(x_ref, wf_ref, wp_ref, z_ref, a_ref):
        zb = jnp.dot(x_ref[...], wf_ref[...], preferred_element_type=jnp.float32)
        z_ref[...] = zb
        a_ref[...] = jnp.dot(zb, wp_ref[...], preferred_element_type=jnp.float32)

    return pl.pallas_call(
        body,
        grid=(10,),
        in_specs=[
            pl.BlockSpec((1000, D), lambda i: (i, 0)),
            pl.BlockSpec((D, D), lambda i: (0, 0)),
            pl.BlockSpec((D, D), lambda i: (0, 0)),
        ],
        out_specs=[
            pl.BlockSpec((1000, D), lambda i: (i, 0)),
            pl.BlockSpec((1000, D), lambda i: (i, 0)),
        ],
        out_shape=[
            jax.ShapeDtypeStruct((N, D), jnp.float32),
            jax.ShapeDtypeStruct((N, D), jnp.float32),
        ],
    )(x, W_fc, Wpair)


@functools.partial(
    pl.kernel,
    out_type=[
        jax.ShapeDtypeStruct((EP,), jnp.float32),
        jax.ShapeDtypeStruct((NW * NPAD,), jnp.float32),
    ],
    mesh=_mesh(),
    compiler_params=pltpu.CompilerParams(needs_layout_passes=False),
    scratch_types=[
        pltpu.VMEM((NPAD,), jnp.float32),
        pltpu.VMEM((NPAD,), jnp.float32),
        pltpu.VMEM((NPAD,), jnp.float32),
        pltpu.VMEM((EPW,), jnp.int32),
        pltpu.VMEM((EPW,), jnp.int32),
        pltpu.VMEM((EPW,), jnp.float32),
        pltpu.VMEM((EPW,), jnp.float32),
    ],
)
def _score(a_s_h, a_d_h, src_h, dst_h, ew_h, p_h, den_h,
           a_s_v, a_d_v, den_v, src_v, dst_v, ew_v, p_v):
    cid = lax.axis_index("c")
    sid = lax.axis_index("s")
    wid = cid * NS + sid
    base = wid * EPW

    pltpu.sync_copy(a_s_h, a_s_v)
    pltpu.sync_copy(a_d_h, a_d_v)
    pltpu.sync_copy(src_h.at[pl.ds(base, EPW)], src_v)
    pltpu.sync_copy(dst_h.at[pl.ds(base, EPW)], dst_v)
    pltpu.sync_copy(ew_h.at[pl.ds(base, EPW)], ew_v)

    zeros = jnp.zeros((16,), jnp.float32)

    def zden_body(i, _):
        den_v[pl.ds(i * 16, 16)] = zeros
        return 0

    lax.fori_loop(0, NPAD // 16, zden_body, 0)

    def edge_body(i, _):
        s16 = src_v[pl.ds(i * 16, 16)]
        d16 = dst_v[pl.ds(i * 16, 16)]
        w16 = ew_v[pl.ds(i * 16, 16)]
        t = plsc.load_gather(a_s_v, [s16]) + plsc.load_gather(a_d_v, [d16])
        e = jnp.where(t >= 0.0, t, t * 0.01) * w16
        p = jnp.exp(e)
        p_v[pl.ds(i * 16, 16)] = p
        plsc.addupdate_scatter(den_v, [d16], p)
        return 0

    lax.fori_loop(0, EPW // 16, edge_body, 0)
    pltpu.sync_copy(p_v, p_h.at[pl.ds(base, EPW)])
    pltpu.sync_copy(den_v, den_h.at[pl.ds(wid * NPAD, NPAD)])


@functools.partial(
    pl.kernel,
    out_type=jax.ShapeDtypeStruct((NC, NPAD, D), jnp.float32),
    mesh=_mesh(),
    compiler_params=pltpu.CompilerParams(needs_layout_passes=False),
    scratch_types=[
        pltpu.VMEM((GB, CH), jnp.int32),
        pltpu.VMEM((GB, CH), jnp.int32),
        pltpu.VMEM((GB, CH), jnp.float32),
        pltpu.VMEM((CH, D), jnp.float32),
        pltpu.VMEM((CH, D), jnp.float32),
        pltpu.VMEM_SHARED((NPAD, D), jnp.float32),
        pltpu.SemaphoreType.DMA,
        pltpu.SemaphoreType.DMA,
    ],
)
def _aggregate(src2_h, dst2_h, p2_h, z_h, hp_h,
               src_v, dst_v, p_v, rows0_v, rows1_v, h_sh, sem0, sem1):
    cid = lax.axis_index("c")
    sid = lax.axis_index("s")
    wid = cid * NS + sid
    rbase = wid * NCH
    rows = (rows0_v, rows1_v)
    sems = (sem0, sem1)

    zeros = jnp.zeros((16,), jnp.float32)

    def zrow_body(j, _):
        for q in range(D // 16):
            rows0_v[j, pl.ds(q * 16, 16)] = zeros
        return 0

    lax.fori_loop(0, CH, zrow_body, 0)

    def zcopy_body(m, _):
        pltpu.sync_copy(rows0_v, h_sh.at[pl.ds(sid * SL + m * CH, CH)])
        return 0

    lax.fori_loop(0, SL // CH, zcopy_body, 0)
    plsc.subcore_barrier()

    def block_body(b, _):
        bb = rbase + b * GB
        pltpu.sync_copy(src2_h.at[pl.ds(bb, GB)], src_v)
        pltpu.sync_copy(dst2_h.at[pl.ds(bb, GB)], dst_v)
        pltpu.sync_copy(p2_h.at[pl.ds(bb, GB)], p_v)

        descs = [None, None]
        descs[0] = pltpu.async_copy(z_h.at[src_v.at[0]], rows[0], sems[0])
        for r in range(GB):
            cur = r % 2
            if r + 1 < GB:
                nxt = (r + 1) % 2
                descs[nxt] = pltpu.async_copy(
                    z_h.at[src_v.at[r + 1]], rows[nxt], sems[nxt]
                )
            descs[cur].wait()
            rbuf = rows[cur]

            def scale_body(i, _):
                for u in range(2):
                    j = i * 2 + u
                    pv = plsc.load_gather(
                        p_v, [jnp.full((16,), r, jnp.int32),
                              jnp.full((16,), j, jnp.int32)]
                    )
                    for q in range(D // 16):
                        rbuf[j, pl.ds(q * 16, 16)] = (
                            rbuf[j, pl.ds(q * 16, 16)] * pv
                        )
                return 0

            lax.fori_loop(0, CH // 2, scale_body, 0)
            pltpu.sync_copy(rbuf, h_sh.at[dst_v.at[r]], add=True)
        return 0

    lax.fori_loop(0, NB, block_body, 0)
    plsc.subcore_barrier()
    pltpu.sync_copy(
        h_sh.at[pl.ds(sid * SL, SL)], hp_h.at[cid, pl.ds(sid * SL, SL)]
    )


def _combine(hp, den):
    def body(hp_ref, den_ref, o_ref):
        dsum = jnp.sum(den_ref[...], axis=0) + 1e-9
        o_ref[...] = (hp_ref[0] + hp_ref[1]) / dsum[:, None]

    return pl.pallas_call(
        body,
        grid=(8,),
        in_specs=[
            pl.BlockSpec((NC, 1280, D), lambda i: (0, i, 0)),
            pl.BlockSpec((NW, 1280), lambda i: (0, i)),
        ],
        out_specs=pl.BlockSpec((1280, D), lambda i: (i, 0)),
        out_shape=jax.ShapeDtypeStruct((NPAD, D), jnp.float32),
    )(hp, den)


def kernel(x, edge_index, edge_weight, W_fc, W_attn):
    src = edge_index[0].astype(jnp.int32)
    dst = edge_index[1].astype(jnp.int32)
    pad = EP - E
    pad_idx = jnp.arange(pad, dtype=jnp.int32)
    src = jnp.concatenate([src, pad_idx % N])
    dst = jnp.concatenate([dst, N + pad_idx % (NPAD - N)])
    ew = jnp.pad(edge_weight, (0, pad))
    wa = W_attn[:, 0]
    Wpair = jnp.zeros((D, D), jnp.float32).at[:, 0].set(wa[:D]).at[:, 1].set(wa[D:])

    z, a = _project(x, W_fc, Wpair)
    a_s = jnp.pad(a[:, 0], (0, NPAD - N))
    a_d = jnp.pad(a[:, 1], (0, NPAD - N))

    p, den = _score(a_s, a_d, src, dst, ew)

    hp = _aggregate(
        src.reshape(EP // CH, CH),
        dst.reshape(EP // CH, CH),
        p.reshape(EP // CH, CH),
        z,
    )
    return _combine(hp, den.reshape(NW, NPAD))[:N]
